# baseline (speedup 1.0000x reference)
import functools

import jax
import jax.numpy as jnp
from jax import lax
from jax.experimental import pallas as pl
from jax.experimental.pallas import tpu as pltpu
from jax.experimental.pallas import tpu_sc as plsc

N = 10000
E = 320000
D = 128
K = 10
ALPHA = 0.1

NC = 2
NS = 16
NW = NC * NS
NP = 10240
EP = 327680
EPT = EP // NW
WIN = 128
NWIN = EPT // WIN
NB = 2
CW = 8
NCH = NWIN // CW
NBI = 2
RPT = NP // NS
ZR = 16


def _mlp_body(x_ref, w1_ref, b1_ref, w2_ref, b2_ref, o_ref):
    h = lax.dot_general(x_ref[...], w1_ref[...], (((1,), (1,)), ((), ())),
                        preferred_element_type=jnp.float32,
                        precision=lax.Precision.HIGHEST)
    h = h + b1_ref[...]
    h = lax.dot_general(h, w2_ref[...], (((1,), (1,)), ((), ())),
                        preferred_element_type=jnp.float32,
                        precision=lax.Precision.HIGHEST)
    o_ref[...] = h + b2_ref[...]


def _mlp(x, W1, b1, W2, b2):
    blk = 1000
    return pl.pallas_call(
        _mlp_body,
        grid=(N // blk,),
        in_specs=[
            pl.BlockSpec((blk, D), lambda i: (i, 0)),
            pl.BlockSpec((D, D), lambda i: (0, 0)),
            pl.BlockSpec((1, D), lambda i: (0, 0)),
            pl.BlockSpec((D, D), lambda i: (0, 0)),
            pl.BlockSpec((1, D), lambda i: (0, 0)),
        ],
        out_specs=pl.BlockSpec((blk, D), lambda i: (i, 0)),
        out_shape=jax.ShapeDtypeStruct((N, D), jnp.float32),
    )(x, W1, b1, W2, b2)


def _sc_round(h, src4, dst4):
    @functools.partial(
        pl.kernel,
        out_type=jax.ShapeDtypeStruct((NC, NP, D), jnp.float32),
        mesh=plsc.VectorSubcoreMesh(core_axis_name="c", subcore_axis_name="s"),
        scratch_types=[
            pltpu.VMEM_SHARED((NP, D), jnp.float32),
            pltpu.VMEM((ZR, D), jnp.float32),
            [pltpu.VMEM((CW, WIN), jnp.int32) for _ in range(NBI)],
            [pltpu.VMEM((CW, WIN), jnp.int32) for _ in range(NBI)],
            [pltpu.VMEM((WIN, D), jnp.float32) for _ in range(NB)],
            [pltpu.SemaphoreType.DMA for _ in range(NBI)],
            [pltpu.SemaphoreType.DMA for _ in range(NB)],
            [pltpu.SemaphoreType.DMA for _ in range(NB)],
            pltpu.SemaphoreType.DMA,
        ],
    )
    def k(h_hbm, src_hbm, dst_hbm, p_hbm, acc, zbuf, sidx, didx, rows,
          isem, gsem, ssem, sem):
        c = lax.axis_index("c")
        s = lax.axis_index("s")
        tile = c * NS + s

        @pl.loop(0, ZR)
        def _(r):
            @pl.loop(0, D, step=16)
            def _(j):
                zbuf.at[r][pl.ds(j, 16)] = jnp.zeros((16,), jnp.float32)

        r0 = s * RPT
        zcopies = [
            pltpu.make_async_copy(zbuf, acc.at[pl.ds(r0 + i * ZR, ZR)], sem)
            for i in range(RPT // ZR)
        ]
        for cp in zcopies:
            cp.start()

        def fire_idx(ch):
            bi = ch % NBI
            pltpu.async_copy(src_hbm.at[tile].at[ch], sidx[bi], isem[bi])
            pltpu.async_copy(dst_hbm.at[tile].at[ch], didx[bi], isem[bi])

        def wait_idx(ch):
            bi = ch % NBI
            pltpu.make_async_copy(src_hbm.at[0].at[0], sidx[bi], isem[bi]).wait()
            pltpu.make_async_copy(dst_hbm.at[0].at[0], didx[bi], isem[bi]).wait()

        def fire_gather(w, b):
            bi = (w // CW) % NBI
            pltpu.async_copy(h_hbm.at[sidx[bi].at[w % CW]], rows[b], gsem[b])

        def wait_gather(b):
            pltpu.make_async_copy(h_hbm.at[sidx[0].at[0]], rows[b],
                                  gsem[b]).wait()

        def fire_scatter(w, b):
            bi = (w // CW) % NBI
            pltpu.async_copy(rows[b], acc.at[didx[bi].at[w % CW]], ssem[b],
                             add=True)

        def wait_scatter(b):
            pltpu.make_async_copy(rows[b], acc.at[didx[0].at[0]], ssem[b]).wait()

        fire_idx(0)
        fire_idx(1)

        for cp in zcopies:
            cp.wait()
        plsc.subcore_barrier()

        idx_ready = set()

        def ensure_idx(ch):
            if ch not in idx_ready:
                wait_idx(ch)
                idx_ready.add(ch)

        for b in range(NB):
            ensure_idx(b // CW)
            fire_gather(b, b)

        for w in range(NWIN):
            ch, wl = divmod(w, CW)
            b = w % NB
            if wl == 0 and ch >= 1 and ch + 1 < NCH:
                fire_idx(ch + 1)
            wait_gather(b)
            fire_scatter(w, b)
            nw = w + NB
            if nw < NWIN:
                wait_scatter(b)
                ensure_idx(nw // CW)
                fire_gather(nw, b)

        for b in range(NB):
            wait_scatter(b)

        plsc.subcore_barrier()

        pltpu.sync_copy(acc.at[pl.ds(r0, RPT)], p_hbm.at[c].at[pl.ds(r0, RPT)])

    return k(h, src4, dst4)


def _combine_body(p_ref, h0_ref, o_ref):
    o_ref[...] = ((1.0 - ALPHA) * (p_ref[0] + p_ref[1])
                  + ALPHA * h0_ref[...])


def _combine(p, h0):
    blk = 1000
    return pl.pallas_call(
        _combine_body,
        grid=(N // blk,),
        in_specs=[
            pl.BlockSpec((NC, blk, D), lambda i: (0, i, 0), ),
            pl.BlockSpec((blk, D), lambda i: (i, 0)),
        ],
        out_specs=pl.BlockSpec((blk, D), lambda i: (i, 0)),
        out_shape=jax.ShapeDtypeStruct((N, D), jnp.float32),
    )(p, h0)


def kernel(x, edge_index, W1, b1, W2, b2):
    pad = EP - E
    src_p = jnp.concatenate([edge_index[0], jnp.zeros((pad,), jnp.int32)])
    dst_p = jnp.concatenate([edge_index[1], jnp.full((pad,), NP - 8, jnp.int32)])
    src4 = src_p.reshape(NW, NCH, CW, WIN)
    dst4 = dst_p.reshape(NW, NCH, CW, WIN)
    h0 = _mlp(x, W1, b1.reshape(1, D), W2, b2.reshape(1, D))
    h = h0
    for _ in range(K):
        p = _sc_round(h, src4, dst4)
        h = _combine(p, h0)
    return h


# speedup vs baseline: 4.1085x; 4.1085x over previous
import functools

import jax
import jax.numpy as jnp
from jax import lax
from jax.experimental import pallas as pl
from jax.experimental.pallas import tpu as pltpu
from jax.experimental.pallas import tpu_sc as plsc

N = 10000
E = 320000
D = 128
K = 10
ALPHA = 0.1

NC = 2
NS = 16
NW = NC * NS
NP = 10240
EP = 327680
EPT = EP // NW
WIN = 128
NWIN = EPT // WIN
NB = 2
CW = 8
NCH = NWIN // CW
NBI = 2
RPT = NP // NS
ZR = 16


def _mlp_body(x_ref, w1_ref, b1_ref, w2_ref, b2_ref, o_ref):
    h = lax.dot_general(x_ref[...], w1_ref[...], (((1,), (1,)), ((), ())),
                        preferred_element_type=jnp.float32,
                        precision=lax.Precision.HIGHEST)
    h = h + b1_ref[...]
    h = lax.dot_general(h, w2_ref[...], (((1,), (1,)), ((), ())),
                        preferred_element_type=jnp.float32,
                        precision=lax.Precision.HIGHEST)
    o_ref[...] = h + b2_ref[...]


def _mlp(x, W1, b1, W2, b2):
    blk = 1000
    return pl.pallas_call(
        _mlp_body,
        grid=(N // blk,),
        in_specs=[
            pl.BlockSpec((blk, D), lambda i: (i, 0)),
            pl.BlockSpec((D, D), lambda i: (0, 0)),
            pl.BlockSpec((1, D), lambda i: (0, 0)),
            pl.BlockSpec((D, D), lambda i: (0, 0)),
            pl.BlockSpec((1, D), lambda i: (0, 0)),
        ],
        out_specs=pl.BlockSpec((blk, D), lambda i: (i, 0)),
        out_shape=jax.ShapeDtypeStruct((N, D), jnp.float32),
    )(x, W1, b1, W2, b2)


def _sc_round(h, src4, dst4):
    @functools.partial(
        pl.kernel,
        out_type=jax.ShapeDtypeStruct((NC, NP, D), jnp.float32),
        mesh=plsc.VectorSubcoreMesh(core_axis_name="c", subcore_axis_name="s"),
        scratch_types=[
            pltpu.VMEM_SHARED((NP, D), jnp.float32),
            pltpu.VMEM((ZR, D), jnp.float32),
            [pltpu.VMEM((CW, WIN), jnp.int32) for _ in range(NBI)],
            [pltpu.VMEM((CW, WIN), jnp.int32) for _ in range(NBI)],
            [pltpu.VMEM((WIN, D), jnp.float32) for _ in range(NB)],
            [pltpu.SemaphoreType.DMA for _ in range(NBI)],
            [pltpu.SemaphoreType.DMA for _ in range(NB)],
            [pltpu.SemaphoreType.DMA for _ in range(NB)],
            pltpu.SemaphoreType.DMA,
        ],
    )
    def k(h_hbm, src_hbm, dst_hbm, p_hbm, acc, zbuf, sidx, didx, rows,
          isem, gsem, ssem, sem):
        c = lax.axis_index("c")
        s = lax.axis_index("s")
        tile = c * NS + s

        @pl.loop(0, ZR)
        def _(r):
            @pl.loop(0, D, step=16)
            def _(j):
                zbuf.at[r][pl.ds(j, 16)] = jnp.zeros((16,), jnp.float32)

        r0 = s * RPT
        zcopies = [
            pltpu.make_async_copy(zbuf, acc.at[pl.ds(r0 + i * ZR, ZR)], sem)
            for i in range(RPT // ZR)
        ]
        for cp in zcopies:
            cp.start()

        def fire_idx(ch):
            bi = ch % NBI
            pltpu.async_copy(src_hbm.at[tile].at[ch], sidx[bi], isem[bi])
            pltpu.async_copy(dst_hbm.at[tile].at[ch], didx[bi], isem[bi])

        def wait_idx(ch):
            bi = ch % NBI
            pltpu.make_async_copy(src_hbm.at[0].at[0], sidx[bi], isem[bi]).wait()
            pltpu.make_async_copy(dst_hbm.at[0].at[0], didx[bi], isem[bi]).wait()

        def fire_gather(w, b):
            bi = (w // CW) % NBI
            pltpu.async_copy(h_hbm.at[sidx[bi].at[w % CW]], rows[b], gsem[b])

        def wait_gather(b):
            pltpu.make_async_copy(h_hbm.at[sidx[0].at[0]], rows[b],
                                  gsem[b]).wait()

        def fire_scatter(w, b):
            bi = (w // CW) % NBI
            pltpu.async_copy(rows[b], acc.at[didx[bi].at[w % CW]], ssem[b],
                             add=True)

        def wait_scatter(b):
            pltpu.make_async_copy(rows[b], acc.at[didx[0].at[0]], ssem[b]).wait()

        fire_idx(0)
        fire_idx(1)

        for cp in zcopies:
            cp.wait()
        plsc.subcore_barrier()

        idx_ready = set()

        def ensure_idx(ch):
            if ch not in idx_ready:
                wait_idx(ch)
                idx_ready.add(ch)

        for b in range(NB):
            ensure_idx(b // CW)
            fire_gather(b, b)

        for w in range(NWIN):
            ch, wl = divmod(w, CW)
            b = w % NB
            if wl == 0 and ch >= 1 and ch + 1 < NCH:
                fire_idx(ch + 1)
            wait_gather(b)
            fire_scatter(w, b)
            nw = w + NB
            if nw < NWIN:
                wait_scatter(b)
                ensure_idx(nw // CW)
                fire_gather(nw, b)

        for b in range(NB):
            wait_scatter(b)

        plsc.subcore_barrier()

        pltpu.sync_copy(acc.at[pl.ds(r0, RPT)], p_hbm.at[c].at[pl.ds(r0, RPT)])

    return k(h, src4, dst4)


def _combine_body(p_ref, h0_ref, o_ref):
    o_ref[...] = ((1.0 - ALPHA) * (p_ref[0] + p_ref[1])
                  + ALPHA * h0_ref[...])


def _combine(p, h0):
    blk = 1000
    return pl.pallas_call(
        _combine_body,
        grid=(N // blk,),
        in_specs=[
            pl.BlockSpec((NC, blk, D), lambda i: (0, i, 0), ),
            pl.BlockSpec((blk, D), lambda i: (i, 0)),
        ],
        out_specs=pl.BlockSpec((blk, D), lambda i: (i, 0)),
        out_shape=jax.ShapeDtypeStruct((N, D), jnp.float32),
    )(p, h0)


def kernel(x, edge_index, W1, b1, W2, b2):
    pad = EP - E
    pad_src = (jnp.arange(pad, dtype=jnp.int32) * 13) % N
    pad_dst = N + (jnp.arange(pad, dtype=jnp.int32) % (NP - N))
    src_p = jnp.concatenate([edge_index[0], pad_src])
    dst_p = jnp.concatenate([edge_index[1], pad_dst])
    src4 = src_p.reshape(NW, NCH, CW, WIN)
    dst4 = dst_p.reshape(NW, NCH, CW, WIN)
    h0 = _mlp(x, W1, b1.reshape(1, D), W2, b2.reshape(1, D))
    h = h0
    for _ in range(K):
        p = _sc_round(h, src4, dst4)
        h = _combine(p, h0)
    return h


# speedup vs baseline: 4.6249x; 1.1257x over previous
import functools

import jax
import jax.numpy as jnp
from jax import lax
from jax.experimental import pallas as pl
from jax.experimental.pallas import tpu as pltpu
from jax.experimental.pallas import tpu_sc as plsc

N = 10000
E = 320000
D = 128
K = 10
ALPHA = 0.1

NC = 2
NS = 16
NW = NC * NS
NP = 10240
EP = 327680
EPT = EP // NW
WIN = 80
NWIN = EPT // WIN
NB = 4
CW = 8
NCH = NWIN // CW
NBI = 2
RPT = NP // NS
ZR = 16


def _mlp_body(x_ref, w1_ref, b1_ref, w2_ref, b2_ref, o_ref):
    h = lax.dot_general(x_ref[...], w1_ref[...], (((1,), (1,)), ((), ())),
                        preferred_element_type=jnp.float32,
                        precision=lax.Precision.HIGHEST)
    h = h + b1_ref[...]
    h = lax.dot_general(h, w2_ref[...], (((1,), (1,)), ((), ())),
                        preferred_element_type=jnp.float32,
                        precision=lax.Precision.HIGHEST)
    o_ref[...] = h + b2_ref[...]


def _mlp(x, W1, b1, W2, b2):
    blk = 1000
    return pl.pallas_call(
        _mlp_body,
        grid=(N // blk,),
        in_specs=[
            pl.BlockSpec((blk, D), lambda i: (i, 0)),
            pl.BlockSpec((D, D), lambda i: (0, 0)),
            pl.BlockSpec((1, D), lambda i: (0, 0)),
            pl.BlockSpec((D, D), lambda i: (0, 0)),
            pl.BlockSpec((1, D), lambda i: (0, 0)),
        ],
        out_specs=pl.BlockSpec((blk, D), lambda i: (i, 0)),
        out_shape=jax.ShapeDtypeStruct((N, D), jnp.float32),
    )(x, W1, b1, W2, b2)


def _sc_round(h, src4, dst4):
    @functools.partial(
        pl.kernel,
        out_type=jax.ShapeDtypeStruct((NC, NP, D), jnp.float32),
        mesh=plsc.VectorSubcoreMesh(core_axis_name="c", subcore_axis_name="s"),
        scratch_types=[
            pltpu.VMEM_SHARED((NP, D), jnp.float32),
            pltpu.VMEM((ZR, D), jnp.float32),
            [pltpu.VMEM((CW, WIN), jnp.int32) for _ in range(NBI)],
            [pltpu.VMEM((CW, WIN), jnp.int32) for _ in range(NBI)],
            [pltpu.VMEM((WIN, D), jnp.float32) for _ in range(NB)],
            [pltpu.SemaphoreType.DMA for _ in range(NBI)],
            [pltpu.SemaphoreType.DMA for _ in range(NB)],
            [pltpu.SemaphoreType.DMA for _ in range(NB)],
            pltpu.SemaphoreType.DMA,
        ],
    )
    def k(h_hbm, src_hbm, dst_hbm, p_hbm, acc, zbuf, sidx, didx, rows,
          isem, gsem, ssem, sem):
        c = lax.axis_index("c")
        s = lax.axis_index("s")
        tile = c * NS + s

        @pl.loop(0, ZR)
        def _(r):
            @pl.loop(0, D, step=16)
            def _(j):
                zbuf.at[r][pl.ds(j, 16)] = jnp.zeros((16,), jnp.float32)

        r0 = s * RPT
        zcopies = [
            pltpu.make_async_copy(zbuf, acc.at[pl.ds(r0 + i * ZR, ZR)], sem)
            for i in range(RPT // ZR)
        ]
        for cp in zcopies:
            cp.start()

        def fire_idx(ch):
            bi = ch % NBI
            pltpu.async_copy(src_hbm.at[tile].at[ch], sidx[bi], isem[bi])
            pltpu.async_copy(dst_hbm.at[tile].at[ch], didx[bi], isem[bi])

        def wait_idx(ch):
            bi = ch % NBI
            pltpu.make_async_copy(src_hbm.at[0].at[0], sidx[bi], isem[bi]).wait()
            pltpu.make_async_copy(dst_hbm.at[0].at[0], didx[bi], isem[bi]).wait()

        def fire_gather(w, b):
            bi = (w // CW) % NBI
            pltpu.async_copy(h_hbm.at[sidx[bi].at[w % CW]], rows[b], gsem[b])

        def wait_gather(b):
            pltpu.make_async_copy(h_hbm.at[sidx[0].at[0]], rows[b],
                                  gsem[b]).wait()

        def fire_scatter(w, b):
            bi = (w // CW) % NBI
            pltpu.async_copy(rows[b], acc.at[didx[bi].at[w % CW]], ssem[b],
                             add=True)

        def wait_scatter(b):
            pltpu.make_async_copy(rows[b], acc.at[didx[0].at[0]], ssem[b]).wait()

        fire_idx(0)
        fire_idx(1)

        for cp in zcopies:
            cp.wait()
        plsc.subcore_barrier()

        idx_ready = set()

        def ensure_idx(ch):
            if ch not in idx_ready:
                wait_idx(ch)
                idx_ready.add(ch)

        for b in range(NB):
            ensure_idx(b // CW)
            fire_gather(b, b)

        for w in range(NWIN):
            ch, wl = divmod(w, CW)
            b = w % NB
            if wl == 0 and ch >= 1 and ch + 1 < NCH:
                fire_idx(ch + 1)
            wait_gather(b)
            fire_scatter(w, b)
            nw = w + NB
            if nw < NWIN:
                wait_scatter(b)
                ensure_idx(nw // CW)
                fire_gather(nw, b)

        for b in range(NB):
            wait_scatter(b)

        plsc.subcore_barrier()

        pltpu.sync_copy(acc.at[pl.ds(r0, RPT)], p_hbm.at[c].at[pl.ds(r0, RPT)])

    return k(h, src4, dst4)


def _combine_body(p_ref, h0_ref, o_ref):
    o_ref[...] = ((1.0 - ALPHA) * (p_ref[0] + p_ref[1])
                  + ALPHA * h0_ref[...])


def _combine(p, h0):
    blk = 1000
    return pl.pallas_call(
        _combine_body,
        grid=(N // blk,),
        in_specs=[
            pl.BlockSpec((NC, blk, D), lambda i: (0, i, 0), ),
            pl.BlockSpec((blk, D), lambda i: (i, 0)),
        ],
        out_specs=pl.BlockSpec((blk, D), lambda i: (i, 0)),
        out_shape=jax.ShapeDtypeStruct((N, D), jnp.float32),
    )(p, h0)


def kernel(x, edge_index, W1, b1, W2, b2):
    pad = EP - E
    pad_src = (jnp.arange(pad, dtype=jnp.int32) * 13) % N
    pad_dst = N + (jnp.arange(pad, dtype=jnp.int32) % (NP - N))
    src_p = jnp.concatenate([edge_index[0], pad_src])
    dst_p = jnp.concatenate([edge_index[1], pad_dst])
    src4 = src_p.reshape(NW, NCH, CW, WIN)
    dst4 = dst_p.reshape(NW, NCH, CW, WIN)
    h0 = _mlp(x, W1, b1.reshape(1, D), W2, b2.reshape(1, D))
    h = h0
    for _ in range(K):
        p = _sc_round(h, src4, dst4)
        h = _combine(p, h0)
    return h
